# baseline (speedup 1.0000x reference)
import functools

import jax
import jax.numpy as jnp
from jax.experimental import pallas as pl
from jax.experimental.pallas import tpu as pltpu
from jax.experimental.pallas import tpu_sc as plsc

_CAP = 64
_LOSS_COEF = 0.01
_SB = 512


def _routing_body(x_ref, w_ref, idx1_ref, idx2_ref, g1_ref, g2_ref,
                  cnt_ref, gsum_ref):
    b = pl.program_id(1)
    x = x_ref[0]
    w = w_ref[...]
    sb = x.shape[0]
    e = w.shape[1]

    logits = jnp.dot(x, w, preferred_element_type=jnp.float32)
    mx = jnp.max(logits, axis=-1, keepdims=True)
    ex = jnp.exp(logits - mx)
    raw = ex / jnp.sum(ex, axis=-1, keepdims=True)

    eidx = jax.lax.broadcasted_iota(jnp.int32, (sb, e), 1)
    m1 = jnp.max(raw, axis=-1, keepdims=True)
    idx1 = jnp.min(jnp.where(raw == m1, eidx, e), axis=-1, keepdims=True)
    oh1 = (eidx == idx1).astype(jnp.float32)
    gate1 = jnp.sum(raw * oh1, axis=-1, keepdims=True)

    raw2 = raw * (1.0 - oh1)
    m2 = jnp.max(raw2, axis=-1, keepdims=True)
    idx2 = jnp.min(jnp.where(raw2 == m2, eidx, e), axis=-1, keepdims=True)
    oh2 = (eidx == idx2).astype(jnp.float32)
    gate2 = jnp.sum(raw * oh2, axis=-1, keepdims=True)

    denom = gate1 + gate2 + 1e-9
    idx1_ref[0] = idx1.T
    idx2_ref[0] = idx2.T
    g1_ref[0] = (gate1 / denom).T
    g2_ref[0] = (gate2 / denom).T

    csum = jnp.sum(oh1, axis=0, keepdims=True)
    gsum = jnp.sum(raw, axis=0, keepdims=True)

    @pl.when(b == 0)
    def _init():
        cnt_ref[0] = csum
        gsum_ref[0] = gsum

    @pl.when(b != 0)
    def _acc():
        cnt_ref[0] += csum
        gsum_ref[0] += gsum


def _emit_body(aux_scale, idx1_ref, idx2_ref, g1_ref, g2_ref, cnt_ref,
               cnt_all_ref, gsum_all_ref, comb_ref, disp_ref, aux_ref,
               c1_scr, c2_scr):
    b = pl.program_id(1)

    @pl.when(b == 0)
    def _reset():
        c1_scr[...] = jnp.zeros_like(c1_scr)
        c2_scr[...] = jnp.zeros_like(c2_scr)

    idx1 = idx1_ref[0]
    idx2 = idx2_ref[0]
    g1 = g1_ref[0]
    g2 = g2_ref[0]
    sb = idx1.shape[1]
    e = c1_scr.shape[0]

    eidx = jax.lax.broadcasted_iota(jnp.int32, (e, sb), 0)
    oh1 = (eidx == idx1).astype(jnp.float32)
    oh2 = (eidx == idx2).astype(jnp.float32)

    r = jax.lax.broadcasted_iota(jnp.int32, (sb, sb), 0)
    c = jax.lax.broadcasted_iota(jnp.int32, (sb, sb), 1)
    triu = (r <= c).astype(jnp.float32)
    cum1 = jnp.dot(oh1, triu, preferred_element_type=jnp.float32)
    cum2 = jnp.dot(oh2, triu, preferred_element_type=jnp.float32)

    c1pre = c1_scr[...]
    c2pre = c2_scr[...]
    cnt1 = jnp.minimum(cnt_ref[0].T, float(_CAP))

    pos1 = cum1 - 1.0 + c1pre
    p1 = jnp.sum(pos1 * oh1, axis=0, keepdims=True)
    keep1 = (p1 < _CAP).astype(jnp.float32)
    pos2 = cum2 - 1.0 + c2pre + cnt1
    p2 = jnp.sum(pos2 * oh2, axis=0, keepdims=True)
    keep2 = (p2 < _CAP).astype(jnp.float32)

    c1_scr[...] = c1pre + cum1[:, sb - 1:sb]
    c2_scr[...] = c2pre + cum2[:, sb - 1:sb]

    cidx = jax.lax.broadcasted_iota(jnp.int32, (_CAP, sb), 0)
    ohc1 = (cidx == p1.astype(jnp.int32)).astype(jnp.float32)
    ohc2 = (cidx == p2.astype(jnp.int32)).astype(jnp.float32)

    t1 = (g1 * keep1) * oh1
    t2 = (g2 * keep2) * oh2
    comb = (t1[:, None, :] * ohc1[None, :, :]
            + t2[:, None, :] * ohc2[None, :, :])
    comb_ref[0] = comb
    disp_ref[0] = (comb > 0.0).astype(jnp.float32)

    prod = gsum_all_ref[...] * cnt_all_ref[...]
    aux_ref[...] = jnp.sum(prod, axis=(0, 2), keepdims=True)[0] * aux_scale


def _sc_zero_fill(dep, g, e, s):
    rows = _CAP // 2
    nw = 32
    per_w = (g * e) // nw

    mesh = plsc.VectorSubcoreMesh(core_axis_name="c", subcore_axis_name="s")

    @functools.partial(
        pl.kernel,
        out_type=jax.ShapeDtypeStruct((g, e, _CAP, s), jnp.float32),
        mesh=mesh,
        scratch_types=[pltpu.VMEM((rows, s), jnp.float32)],
    )
    def body(dep_hbm, out_hbm, zbuf):
        wid = jax.lax.axis_index("s") * 2 + jax.lax.axis_index("c")

        def zrow(i, _):
            def zchunk(j, _):
                zbuf[i, pl.ds(j * 16, 16)] = jnp.zeros((16,), jnp.float32)
                return 0
            return jax.lax.fori_loop(0, s // 16, zchunk, 0)

        jax.lax.fori_loop(0, rows, zrow, 0)

        def plane(p, _):
            pid = wid * per_w + p
            gi = pid // e
            ei = pid % e
            pltpu.sync_copy(zbuf, out_hbm.at[gi, ei, pl.ds(0, rows)])
            pltpu.sync_copy(zbuf, out_hbm.at[gi, ei, pl.ds(rows, rows)])
            return 0

        jax.lax.fori_loop(0, per_w, plane, 0)

    return body(dep)


def _moe_gating(inputs, gating_weight):
    g, s, m = inputs.shape
    e = gating_weight.shape[1]
    nb = s // _SB

    tok_shape = (g * nb, 1, _SB)
    routing = pl.pallas_call(
        _routing_body,
        grid=(g, nb),
        in_specs=[
            pl.BlockSpec((1, _SB, m), lambda gi, bi: (gi, bi, 0)),
            pl.BlockSpec((m, e), lambda gi, bi: (0, 0)),
        ],
        out_specs=[
            pl.BlockSpec((1, 1, _SB), lambda gi, bi, nb=nb: (gi * nb + bi, 0, 0)),
            pl.BlockSpec((1, 1, _SB), lambda gi, bi, nb=nb: (gi * nb + bi, 0, 0)),
            pl.BlockSpec((1, 1, _SB), lambda gi, bi, nb=nb: (gi * nb + bi, 0, 0)),
            pl.BlockSpec((1, 1, _SB), lambda gi, bi, nb=nb: (gi * nb + bi, 0, 0)),
            pl.BlockSpec((1, 1, e), lambda gi, bi: (gi, 0, 0)),
            pl.BlockSpec((1, 1, e), lambda gi, bi: (gi, 0, 0)),
        ],
        out_shape=[
            jax.ShapeDtypeStruct(tok_shape, jnp.int32),
            jax.ShapeDtypeStruct(tok_shape, jnp.int32),
            jax.ShapeDtypeStruct(tok_shape, jnp.float32),
            jax.ShapeDtypeStruct(tok_shape, jnp.float32),
            jax.ShapeDtypeStruct((g, 1, e), jnp.float32),
            jax.ShapeDtypeStruct((g, 1, e), jnp.float32),
        ],
        compiler_params=pltpu.CompilerParams(
            dimension_semantics=("parallel", "arbitrary")),
    )
    idx1, idx2, g1n, g2n, cnt, gsum = routing(inputs, gating_weight)

    aux_scale = _LOSS_COEF * e / (g * s * s)
    emit = pl.pallas_call(
        functools.partial(_emit_body, aux_scale),
        grid=(g, nb),
        in_specs=[
            pl.BlockSpec((1, 1, _SB), lambda gi, bi, nb=nb: (gi * nb + bi, 0, 0)),
            pl.BlockSpec((1, 1, _SB), lambda gi, bi, nb=nb: (gi * nb + bi, 0, 0)),
            pl.BlockSpec((1, 1, _SB), lambda gi, bi, nb=nb: (gi * nb + bi, 0, 0)),
            pl.BlockSpec((1, 1, _SB), lambda gi, bi, nb=nb: (gi * nb + bi, 0, 0)),
            pl.BlockSpec((1, 1, e), lambda gi, bi: (gi, 0, 0)),
            pl.BlockSpec((g, 1, e), lambda gi, bi: (0, 0, 0)),
            pl.BlockSpec((g, 1, e), lambda gi, bi: (0, 0, 0)),
        ],
        out_specs=[
            pl.BlockSpec((1, e, _CAP, _SB), lambda gi, bi: (gi, 0, 0, bi)),
            pl.BlockSpec((1, e, _CAP, _SB), lambda gi, bi: (gi, 0, 0, bi)),
            pl.BlockSpec((1, 1), lambda gi, bi: (0, 0)),
        ],
        out_shape=[
            jax.ShapeDtypeStruct((g, e, _CAP, s), jnp.float32),
            jax.ShapeDtypeStruct((g, e, _CAP, s), jnp.float32),
            jax.ShapeDtypeStruct((1, 1), jnp.float32),
        ],
        scratch_shapes=[
            pltpu.VMEM((e, 1), jnp.float32),
            pltpu.VMEM((e, 1), jnp.float32),
        ],
        compiler_params=pltpu.CompilerParams(
            dimension_semantics=("parallel", "arbitrary")),
    )
    combT, dispT, aux = emit(idx1, idx2, g1n, g2n, cnt, cnt, gsum)
    comb = jnp.transpose(combT, (0, 3, 1, 2))
    disp = jnp.transpose(dispT, (0, 3, 1, 2))
    scjunk = _sc_zero_fill(gating_weight, g, e, s)
    return comb, disp, aux[0, 0], scjunk[0, 0, 0, 0]


def kernel(inputs, gating_weight, total_token_num):
    del total_token_num
    return _moe_gating(inputs, gating_weight)


# speedup vs baseline: 1.2832x; 1.2832x over previous
import functools

import jax
import jax.numpy as jnp
from jax.experimental import pallas as pl
from jax.experimental.pallas import tpu as pltpu

_CAP = 64
_LOSS_COEF = 0.01
_SB = 256


def _fused_body(g_total, nb, aux_scale,
                x_ref, w_ref, comb_ref, disp_ref, aux_ref,
                idx1_s, idx2_s, g1_s, g2_s, cnt_s, gsum_s,
                c1_scr, c2_scr, aux_acc):
    go = pl.program_id(0)
    b = pl.program_id(1)
    e = w_ref.shape[1]
    sb = _SB

    @pl.when(jnp.logical_and(go == 0, b == 0))
    def _init_aux():
        aux_acc[...] = jnp.zeros_like(aux_acc)

    @pl.when(go < g_total)
    def _routing():
        p = jax.lax.rem(go, 2)
        x = x_ref[0]
        w = w_ref[...]

        logits = jnp.dot(x, w, preferred_element_type=jnp.float32)
        mx = jnp.max(logits, axis=-1, keepdims=True)
        ex = jnp.exp(logits - mx)
        raw = ex / jnp.sum(ex, axis=-1, keepdims=True)

        eidx = jax.lax.broadcasted_iota(jnp.int32, (sb, e), 1)
        m1 = jnp.max(raw, axis=-1, keepdims=True)
        idx1 = jnp.min(jnp.where(raw == m1, eidx, e), axis=-1, keepdims=True)
        oh1 = (eidx == idx1).astype(jnp.float32)
        gate1 = jnp.sum(raw * oh1, axis=-1, keepdims=True)

        raw2 = raw * (1.0 - oh1)
        m2 = jnp.max(raw2, axis=-1, keepdims=True)
        idx2 = jnp.min(jnp.where(raw2 == m2, eidx, e), axis=-1, keepdims=True)
        oh2 = (eidx == idx2).astype(jnp.float32)
        gate2 = jnp.sum(raw * oh2, axis=-1, keepdims=True)

        denom = gate1 + gate2 + 1e-9
        idx1_s[p, b] = idx1.T
        idx2_s[p, b] = idx2.T
        g1_s[p, b] = (gate1 / denom).T
        g2_s[p, b] = (gate2 / denom).T

        csum = jnp.sum(oh1, axis=0, keepdims=True)
        gsum = jnp.sum(raw, axis=0, keepdims=True)

        @pl.when(b == 0)
        def _first():
            cnt_s[p] = csum
            gsum_s[...] = gsum

        @pl.when(b != 0)
        def _acc():
            cnt_s[p] += csum
            gsum_s[...] += gsum

        @pl.when(b == nb - 1)
        def _aux_contrib():
            aux_acc[...] += jnp.sum(gsum_s[...] * cnt_s[p],
                                    axis=(0, 1), keepdims=True)

    @pl.when(go >= 1)
    def _emit():
        pe = jax.lax.rem(go - 1, 2)

        @pl.when(b == 0)
        def _reset():
            c1_scr[...] = jnp.zeros_like(c1_scr)
            c2_scr[...] = jnp.zeros_like(c2_scr)

        idx1 = idx1_s[pe, b]
        idx2 = idx2_s[pe, b]
        g1 = g1_s[pe, b]
        g2 = g2_s[pe, b]

        eidx = jax.lax.broadcasted_iota(jnp.int32, (e, sb), 0)
        oh1 = (eidx == idx1).astype(jnp.float32)
        oh2 = (eidx == idx2).astype(jnp.float32)

        r = jax.lax.broadcasted_iota(jnp.int32, (sb, sb), 0)
        c = jax.lax.broadcasted_iota(jnp.int32, (sb, sb), 1)
        triu = (r <= c).astype(jnp.float32)
        cum1 = jnp.dot(oh1, triu, preferred_element_type=jnp.float32)
        cum2 = jnp.dot(oh2, triu, preferred_element_type=jnp.float32)

        c1pre = c1_scr[...]
        c2pre = c2_scr[...]
        cnt1 = jnp.minimum(cnt_s[pe].T, float(_CAP))

        pos1 = cum1 - 1.0 + c1pre
        p1 = jnp.sum(pos1 * oh1, axis=0, keepdims=True)
        keep1 = (p1 < _CAP).astype(jnp.float32)
        pos2 = cum2 - 1.0 + c2pre + cnt1
        p2 = jnp.sum(pos2 * oh2, axis=0, keepdims=True)
        keep2 = (p2 < _CAP).astype(jnp.float32)

        c1_scr[...] = c1pre + cum1[:, sb - 1:sb]
        c2_scr[...] = c2pre + cum2[:, sb - 1:sb]

        cidx = jax.lax.broadcasted_iota(jnp.int32, (_CAP, sb), 0)
        ohc1 = (cidx == p1.astype(jnp.int32)).astype(jnp.float32)
        ohc2 = (cidx == p2.astype(jnp.int32)).astype(jnp.float32)

        t1 = (g1 * keep1) * oh1
        t2 = (g2 * keep2) * oh2
        comb = (t1[:, None, :] * ohc1[None, :, :]
                + t2[:, None, :] * ohc2[None, :, :])
        comb_ref[0] = comb
        disp_ref[0] = (comb > 0.0).astype(jnp.float32)

    aux_ref[...] = aux_acc[...] * aux_scale


def _moe_gating(inputs, gating_weight):
    g, s, m = inputs.shape
    e = gating_weight.shape[1]
    nb = s // _SB
    aux_scale = _LOSS_COEF * e / (g * s * s)

    def x_map(go, bi, g=g, nb=nb):
        return (jnp.minimum(go, g - 1), jnp.where(go < g, bi, nb - 1), 0)

    def out_map(go, bi):
        return (jnp.maximum(go - 1, 0), 0, 0, jnp.where(go >= 1, bi, 0))

    fused = pl.pallas_call(
        functools.partial(_fused_body, g, nb, aux_scale),
        grid=(g + 1, nb),
        in_specs=[
            pl.BlockSpec((1, _SB, m), x_map),
            pl.BlockSpec((m, e), lambda go, bi: (0, 0)),
        ],
        out_specs=[
            pl.BlockSpec((1, e, _CAP, _SB), out_map),
            pl.BlockSpec((1, e, _CAP, _SB), out_map),
            pl.BlockSpec((1, 1), lambda go, bi: (0, 0)),
        ],
        out_shape=[
            jax.ShapeDtypeStruct((g, e, _CAP, s), jnp.float32),
            jax.ShapeDtypeStruct((g, e, _CAP, s), jnp.float32),
            jax.ShapeDtypeStruct((1, 1), jnp.float32),
        ],
        scratch_shapes=[
            pltpu.VMEM((2, nb, 1, _SB), jnp.int32),
            pltpu.VMEM((2, nb, 1, _SB), jnp.int32),
            pltpu.VMEM((2, nb, 1, _SB), jnp.float32),
            pltpu.VMEM((2, nb, 1, _SB), jnp.float32),
            pltpu.VMEM((2, 1, e), jnp.float32),
            pltpu.VMEM((1, e), jnp.float32),
            pltpu.VMEM((e, 1), jnp.float32),
            pltpu.VMEM((e, 1), jnp.float32),
            pltpu.VMEM((1, 1), jnp.float32),
        ],
        compiler_params=pltpu.CompilerParams(
            dimension_semantics=("arbitrary", "arbitrary")),
    )
    combT, dispT, aux = fused(inputs, gating_weight)
    comb = jnp.transpose(combT, (0, 3, 1, 2))
    disp = jnp.transpose(dispT, (0, 3, 1, 2))
    return comb, disp, aux[0, 0]


def kernel(inputs, gating_weight, total_token_num):
    del total_token_num
    return _moe_gating(inputs, gating_weight)


# speedup vs baseline: 1.2866x; 1.0026x over previous
import functools

import jax
import jax.numpy as jnp
from jax.experimental import pallas as pl
from jax.experimental.pallas import tpu as pltpu

_CAP = 64
_LOSS_COEF = 0.01
_SB = 256


def _fused_body(g_total, nb, aux_scale,
                x_ref, w_ref, comb_ref, disp_ref, aux_ref,
                idx1_s, idx2_s, g1_s, g2_s, cnt_s, gsum_s,
                c1_scr, c2_scr, aux_acc):
    go = pl.program_id(0)
    b = pl.program_id(1)
    e = w_ref.shape[1]
    sb = _SB

    @pl.when(jnp.logical_and(go == 0, b == 0))
    def _init_aux():
        aux_acc[...] = jnp.zeros_like(aux_acc)

    @pl.when(go < g_total)
    def _routing():
        p = jax.lax.rem(go, 2)
        x = x_ref[0]
        w = w_ref[...]

        logits = jnp.dot(x, w, preferred_element_type=jnp.float32)
        mx = jnp.max(logits, axis=-1, keepdims=True)
        ex = jnp.exp(logits - mx)
        raw = ex / jnp.sum(ex, axis=-1, keepdims=True)

        eidx = jax.lax.broadcasted_iota(jnp.int32, (sb, e), 1)
        m1 = jnp.max(raw, axis=-1, keepdims=True)
        idx1 = jnp.min(jnp.where(raw == m1, eidx, e), axis=-1, keepdims=True)
        oh1 = (eidx == idx1).astype(jnp.float32)
        gate1 = jnp.sum(raw * oh1, axis=-1, keepdims=True)

        raw2 = raw * (1.0 - oh1)
        m2 = jnp.max(raw2, axis=-1, keepdims=True)
        idx2 = jnp.min(jnp.where(raw2 == m2, eidx, e), axis=-1, keepdims=True)
        oh2 = (eidx == idx2).astype(jnp.float32)
        gate2 = jnp.sum(raw * oh2, axis=-1, keepdims=True)

        denom = gate1 + gate2 + 1e-9
        idx1_s[p, b] = idx1.T
        idx2_s[p, b] = idx2.T
        g1_s[p, b] = (gate1 / denom).T
        g2_s[p, b] = (gate2 / denom).T

        csum = jnp.sum(oh1, axis=0, keepdims=True)
        gsum = jnp.sum(raw, axis=0, keepdims=True)

        @pl.when(b == 0)
        def _first():
            cnt_s[p] = csum
            gsum_s[...] = gsum

        @pl.when(b != 0)
        def _acc():
            cnt_s[p] += csum
            gsum_s[...] += gsum

        @pl.when(b == nb - 1)
        def _aux_contrib():
            aux_acc[...] += jnp.sum(gsum_s[...] * cnt_s[p],
                                    axis=(0, 1), keepdims=True)

    @pl.when(go >= 1)
    def _emit():
        pe = jax.lax.rem(go - 1, 2)

        @pl.when(b == 0)
        def _reset():
            c1_scr[...] = jnp.zeros_like(c1_scr)
            c2_scr[...] = jnp.zeros_like(c2_scr)

        idx1 = idx1_s[pe, b]
        idx2 = idx2_s[pe, b]
        g1 = g1_s[pe, b]
        g2 = g2_s[pe, b]

        eidx = jax.lax.broadcasted_iota(jnp.int32, (e, sb), 0)
        oh1 = (eidx == idx1).astype(jnp.float32)
        oh2 = (eidx == idx2).astype(jnp.float32)

        r = jax.lax.broadcasted_iota(jnp.int32, (sb, sb), 0)
        c = jax.lax.broadcasted_iota(jnp.int32, (sb, sb), 1)
        triu = (r <= c).astype(jnp.float32)
        cum1 = jnp.dot(oh1, triu, preferred_element_type=jnp.float32)
        cum2 = jnp.dot(oh2, triu, preferred_element_type=jnp.float32)

        c1pre = c1_scr[...]
        c2pre = c2_scr[...]
        cnt1 = jnp.minimum(cnt_s[pe].T, float(_CAP))

        pos1 = cum1 - 1.0 + c1pre
        p1 = jnp.sum(pos1 * oh1, axis=0, keepdims=True)
        keep1 = (p1 < _CAP).astype(jnp.float32)
        pos2 = cum2 - 1.0 + c2pre + cnt1
        p2 = jnp.sum(pos2 * oh2, axis=0, keepdims=True)
        keep2 = (p2 < _CAP).astype(jnp.float32)

        c1_scr[...] = c1pre + cum1[:, sb - 1:sb]
        c2_scr[...] = c2pre + cum2[:, sb - 1:sb]

        cidx = jax.lax.broadcasted_iota(jnp.int32, (_CAP, sb), 0)
        ohc1 = (cidx == p1.astype(jnp.int32)).astype(jnp.float32)
        ohc2 = (cidx == p2.astype(jnp.int32)).astype(jnp.float32)

        k1 = keep1 * oh1
        k2 = keep2 * oh2
        d = (k1[:, None, :] * ohc1[None, :, :]
             + k2[:, None, :] * ohc2[None, :, :])
        gv = g1 * k1 + g2 * k2
        comb_ref[0] = d * gv[:, None, :]
        disp_ref[0] = d

    aux_ref[...] = aux_acc[...] * aux_scale


def _moe_gating(inputs, gating_weight):
    g, s, m = inputs.shape
    e = gating_weight.shape[1]
    nb = s // _SB
    aux_scale = _LOSS_COEF * e / (g * s * s)

    def x_map(go, bi, g=g, nb=nb):
        return (jnp.minimum(go, g - 1), jnp.where(go < g, bi, nb - 1), 0)

    def out_map(go, bi):
        return (jnp.maximum(go - 1, 0), 0, 0, jnp.where(go >= 1, bi, 0))

    fused = pl.pallas_call(
        functools.partial(_fused_body, g, nb, aux_scale),
        grid=(g + 1, nb),
        in_specs=[
            pl.BlockSpec((1, _SB, m), x_map),
            pl.BlockSpec((m, e), lambda go, bi: (0, 0)),
        ],
        out_specs=[
            pl.BlockSpec((1, e, _CAP, _SB), out_map),
            pl.BlockSpec((1, e, _CAP, _SB), out_map),
            pl.BlockSpec((1, 1), lambda go, bi: (0, 0)),
        ],
        out_shape=[
            jax.ShapeDtypeStruct((g, e, _CAP, s), jnp.float32),
            jax.ShapeDtypeStruct((g, e, _CAP, s), jnp.float32),
            jax.ShapeDtypeStruct((1, 1), jnp.float32),
        ],
        scratch_shapes=[
            pltpu.VMEM((2, nb, 1, _SB), jnp.int32),
            pltpu.VMEM((2, nb, 1, _SB), jnp.int32),
            pltpu.VMEM((2, nb, 1, _SB), jnp.float32),
            pltpu.VMEM((2, nb, 1, _SB), jnp.float32),
            pltpu.VMEM((2, 1, e), jnp.float32),
            pltpu.VMEM((1, e), jnp.float32),
            pltpu.VMEM((e, 1), jnp.float32),
            pltpu.VMEM((e, 1), jnp.float32),
            pltpu.VMEM((1, 1), jnp.float32),
        ],
        compiler_params=pltpu.CompilerParams(
            dimension_semantics=("arbitrary", "arbitrary")),
    )
    combT, dispT, aux = fused(inputs, gating_weight)
    comb = jnp.transpose(combT, (0, 3, 1, 2))
    disp = jnp.transpose(dispT, (0, 3, 1, 2))
    return comb, disp, aux[0, 0]


def kernel(inputs, gating_weight, total_token_num):
    del total_token_num
    return _moe_gating(inputs, gating_weight)


# speedup vs baseline: 1.3716x; 1.0661x over previous
import functools

import jax
import jax.numpy as jnp
from jax.experimental import pallas as pl
from jax.experimental.pallas import tpu as pltpu

_CAP = 64
_LOSS_COEF = 0.01
_SBR = 1024
_SBE = 512


def _routing_body(x_ref, w_ref, idx1_ref, idx2_ref, g1_ref, g2_ref,
                  cnt_ref, gsum_ref):
    b = pl.program_id(1)
    x = x_ref[0]
    w = w_ref[...]
    sb = x.shape[0]
    e = w.shape[1]

    logits = jnp.dot(x, w, preferred_element_type=jnp.float32)
    mx = jnp.max(logits, axis=-1, keepdims=True)
    ex = jnp.exp(logits - mx)
    raw = ex / jnp.sum(ex, axis=-1, keepdims=True)

    eidx = jax.lax.broadcasted_iota(jnp.int32, (sb, e), 1)
    m1 = jnp.max(raw, axis=-1, keepdims=True)
    idx1 = jnp.min(jnp.where(raw == m1, eidx, e), axis=-1, keepdims=True)
    oh1 = (eidx == idx1).astype(jnp.float32)
    gate1 = jnp.sum(raw * oh1, axis=-1, keepdims=True)

    raw2 = raw * (1.0 - oh1)
    m2 = jnp.max(raw2, axis=-1, keepdims=True)
    idx2 = jnp.min(jnp.where(raw2 == m2, eidx, e), axis=-1, keepdims=True)
    oh2 = (eidx == idx2).astype(jnp.float32)
    gate2 = jnp.sum(raw * oh2, axis=-1, keepdims=True)

    denom = gate1 + gate2 + 1e-9
    idx1_ref[0] = idx1.T
    idx2_ref[0] = idx2.T
    g1_ref[0] = (gate1 / denom).T
    g2_ref[0] = (gate2 / denom).T

    csum = jnp.sum(oh1, axis=0, keepdims=True)
    gsum = jnp.sum(raw, axis=0, keepdims=True)

    @pl.when(b == 0)
    def _init():
        cnt_ref[0] = csum
        gsum_ref[0] = gsum

    @pl.when(b != 0)
    def _acc():
        cnt_ref[0] += csum
        gsum_ref[0] += gsum


def _emit_body(aux_scale, idx1_ref, idx2_ref, g1_ref, g2_ref, cnt_ref,
               cnt_all_ref, gsum_all_ref, comb_ref, disp_ref, aux_ref,
               c1_scr, c2_scr):
    b = pl.program_id(1)

    @pl.when(b == 0)
    def _reset():
        c1_scr[...] = jnp.zeros_like(c1_scr)
        c2_scr[...] = jnp.zeros_like(c2_scr)

    idx1 = idx1_ref[0]
    idx2 = idx2_ref[0]
    g1 = g1_ref[0]
    g2 = g2_ref[0]
    sb = idx1.shape[1]
    e = c1_scr.shape[0]

    eidx = jax.lax.broadcasted_iota(jnp.int32, (e, sb), 0)
    oh1 = (eidx == idx1).astype(jnp.float32)
    oh2 = (eidx == idx2).astype(jnp.float32)

    r = jax.lax.broadcasted_iota(jnp.int32, (sb, sb), 0)
    c = jax.lax.broadcasted_iota(jnp.int32, (sb, sb), 1)
    triu = (r <= c).astype(jnp.float32)
    cum1 = jnp.dot(oh1, triu, preferred_element_type=jnp.float32)
    cum2 = jnp.dot(oh2, triu, preferred_element_type=jnp.float32)

    c1pre = c1_scr[...]
    c2pre = c2_scr[...]
    cnt1 = jnp.minimum(cnt_ref[0].T, float(_CAP))

    pos1 = cum1 - 1.0 + c1pre
    p1 = jnp.sum(pos1 * oh1, axis=0, keepdims=True)
    keep1 = (p1 < _CAP).astype(jnp.float32)
    pos2 = cum2 - 1.0 + c2pre + cnt1
    p2 = jnp.sum(pos2 * oh2, axis=0, keepdims=True)
    keep2 = (p2 < _CAP).astype(jnp.float32)

    c1_scr[...] = c1pre + cum1[:, sb - 1:sb]
    c2_scr[...] = c2pre + cum2[:, sb - 1:sb]

    cidx = jax.lax.broadcasted_iota(jnp.int32, (_CAP, sb), 0)
    ohc1 = (cidx == p1.astype(jnp.int32)).astype(jnp.float32)
    ohc2 = (cidx == p2.astype(jnp.int32)).astype(jnp.float32)

    k1 = keep1 * oh1
    k2 = keep2 * oh2
    d = (k1[:, None, :] * ohc1[None, :, :]
         + k2[:, None, :] * ohc2[None, :, :])
    gv = g1 * k1 + g2 * k2
    comb_ref[0] = d * gv[:, None, :]
    disp_ref[0] = d

    prod = gsum_all_ref[...] * cnt_all_ref[...]
    aux_ref[...] = jnp.sum(prod, axis=(0, 2), keepdims=True)[0] * aux_scale


def _moe_gating(inputs, gating_weight):
    g, s, m = inputs.shape
    e = gating_weight.shape[1]
    nbr = s // _SBR
    nbe = s // _SBE

    tok_shape = (g * nbr, 1, _SBR)
    routing = pl.pallas_call(
        _routing_body,
        grid=(g, nbr),
        in_specs=[
            pl.BlockSpec((1, _SBR, m), lambda gi, bi: (gi, bi, 0)),
            pl.BlockSpec((m, e), lambda gi, bi: (0, 0)),
        ],
        out_specs=[
            pl.BlockSpec((1, 1, _SBR), lambda gi, bi, nb=nbr: (gi * nb + bi, 0, 0)),
            pl.BlockSpec((1, 1, _SBR), lambda gi, bi, nb=nbr: (gi * nb + bi, 0, 0)),
            pl.BlockSpec((1, 1, _SBR), lambda gi, bi, nb=nbr: (gi * nb + bi, 0, 0)),
            pl.BlockSpec((1, 1, _SBR), lambda gi, bi, nb=nbr: (gi * nb + bi, 0, 0)),
            pl.BlockSpec((1, 1, e), lambda gi, bi: (gi, 0, 0)),
            pl.BlockSpec((1, 1, e), lambda gi, bi: (gi, 0, 0)),
        ],
        out_shape=[
            jax.ShapeDtypeStruct(tok_shape, jnp.int32),
            jax.ShapeDtypeStruct(tok_shape, jnp.int32),
            jax.ShapeDtypeStruct(tok_shape, jnp.float32),
            jax.ShapeDtypeStruct(tok_shape, jnp.float32),
            jax.ShapeDtypeStruct((g, 1, e), jnp.float32),
            jax.ShapeDtypeStruct((g, 1, e), jnp.float32),
        ],
        compiler_params=pltpu.CompilerParams(
            dimension_semantics=("parallel", "arbitrary")),
    )
    idx1, idx2, g1n, g2n, cnt, gsum = routing(inputs, gating_weight)
    tok_e = (g * nbe, 1, _SBE)
    idx1, idx2, g1n, g2n = (a.reshape(tok_e) for a in (idx1, idx2, g1n, g2n))

    aux_scale = _LOSS_COEF * e / (g * s * s)
    emit = pl.pallas_call(
        functools.partial(_emit_body, aux_scale),
        grid=(g, nbe),
        in_specs=[
            pl.BlockSpec((1, 1, _SBE), lambda gi, bi, nb=nbe: (gi * nb + bi, 0, 0)),
            pl.BlockSpec((1, 1, _SBE), lambda gi, bi, nb=nbe: (gi * nb + bi, 0, 0)),
            pl.BlockSpec((1, 1, _SBE), lambda gi, bi, nb=nbe: (gi * nb + bi, 0, 0)),
            pl.BlockSpec((1, 1, _SBE), lambda gi, bi, nb=nbe: (gi * nb + bi, 0, 0)),
            pl.BlockSpec((1, 1, e), lambda gi, bi: (gi, 0, 0)),
            pl.BlockSpec((g, 1, e), lambda gi, bi: (0, 0, 0)),
            pl.BlockSpec((g, 1, e), lambda gi, bi: (0, 0, 0)),
        ],
        out_specs=[
            pl.BlockSpec((1, e, _CAP, _SBE), lambda gi, bi: (gi, 0, 0, bi)),
            pl.BlockSpec((1, e, _CAP, _SBE), lambda gi, bi: (gi, 0, 0, bi)),
            pl.BlockSpec((1, 1), lambda gi, bi: (0, 0)),
        ],
        out_shape=[
            jax.ShapeDtypeStruct((g, e, _CAP, s), jnp.float32),
            jax.ShapeDtypeStruct((g, e, _CAP, s), jnp.float32),
            jax.ShapeDtypeStruct((1, 1), jnp.float32),
        ],
        scratch_shapes=[
            pltpu.VMEM((e, 1), jnp.float32),
            pltpu.VMEM((e, 1), jnp.float32),
        ],
        compiler_params=pltpu.CompilerParams(
            dimension_semantics=("parallel", "arbitrary")),
    )
    combT, dispT, aux = emit(idx1, idx2, g1n, g2n, cnt, cnt, gsum)
    comb = jnp.transpose(combT, (0, 3, 1, 2))
    disp = jnp.transpose(dispT, (0, 3, 1, 2))
    return comb, disp, aux[0, 0]


def kernel(inputs, gating_weight, total_token_num):
    del total_token_num
    return _moe_gating(inputs, gating_weight)
